# baseline (speedup 1.0000x reference)
import functools

import jax
import jax.numpy as jnp
from jax import lax
from jax.experimental import pallas as pl
from jax.experimental.pallas import tpu as pltpu
from jax.experimental.pallas import tpu_sc as plsc

B = 4096
L = 200
D = 64
ROWS = B * L
NC, NS = 2, 16
NW = NC * NS
ROWS_PER_W = ROWS // NW
SUB = 80
NSUB = 10
CHUNK = SUB * NSUB
CHUNKS = ROWS_PER_W // CHUNK


def _body(ids_hbm, table_hbm, pos_hbm, out_hbm, idx_v, rows_v, pos_v, sem):
    cid = lax.axis_index("c")
    sid = lax.axis_index("s")
    wid = sid * NC + cid

    pltpu.sync_copy(pos_hbm.at[pl.ds(0, L)], pos_v)

    def chunk_body(t, carry):
        gbase = wid * ROWS_PER_W + t * CHUNK
        pltpu.sync_copy(ids_hbm.at[pl.ds(gbase, CHUNK)], idx_v)
        copies = []
        for j in range(NSUB):
            copies.append(
                pltpu.async_copy(
                    table_hbm.at[idx_v.at[pl.ds(j * SUB, SUB)]],
                    rows_v.at[pl.ds(j * SUB, SUB)],
                    sem,
                )
            )
        for c in copies:
            c.wait()

        def add_body(pr, c2):
            for q in range(D // 16):
                pv = pos_v[pr, pl.ds(q * 16, 16)]
                for rep in range(CHUNK // L):
                    plsc.addupdate(rows_v.at[rep * L + pr, pl.ds(q * 16, 16)], pv)
            return c2

        lax.fori_loop(0, L, add_body, 0)

        pltpu.sync_copy(rows_v, out_hbm.at[pl.ds(gbase, CHUNK)])
        return carry

    lax.fori_loop(0, CHUNKS, chunk_body, 0)


@jax.jit
def _run(ids2d, token_table, pos_table):
    mesh = plsc.VectorSubcoreMesh(
        core_axis_name="c", subcore_axis_name="s", num_cores=NC, num_subcores=NS
    )
    f = pl.kernel(
        _body,
        out_type=jax.ShapeDtypeStruct((ROWS, D), jnp.float32),
        mesh=mesh,
        scratch_types=[
            pltpu.VMEM((CHUNK,), jnp.int32),
            pltpu.VMEM((CHUNK, D), jnp.float32),
            pltpu.VMEM((L, D), jnp.float32),
            pltpu.SemaphoreType.DMA,
        ],
        compiler_params=pltpu.CompilerParams(use_tc_tiling_on_sc=False),
    )
    return f(ids2d, token_table, pos_table)


def kernel(input_ids, token_table, pos_table):
    ids_flat = input_ids.astype(jnp.int32).reshape(ROWS)
    out = _run(ids_flat, token_table, pos_table)
    return out.reshape(B, L, D)


# speedup vs baseline: 1.0773x; 1.0773x over previous
import jax
import jax.numpy as jnp
from jax import lax
from jax.experimental import pallas as pl
from jax.experimental.pallas import tpu as pltpu
from jax.experimental.pallas import tpu_sc as plsc

B = 4096
L = 200
D = 64
ROWS = B * L
NC, NS = 2, 16
NW = NC * NS
ROWS_PER_W = ROWS // NW
SUB = 80
NSUB = 5
CHUNK = SUB * NSUB
CHUNKS = ROWS_PER_W // CHUNK
NBUF = 4


def _body(ids_hbm, table_hbm, pos_hbm, out_hbm, idxs, rows, pos_v, gsems, ssems):
    cid = lax.axis_index("c")
    sid = lax.axis_index("s")
    wid = sid * NC + cid
    wbase = wid * ROWS_PER_W

    pltpu.sync_copy(pos_hbm.at[pl.ds(0, L)], pos_v)

    def fire(t, b):
        base = wbase + t * CHUNK
        pltpu.sync_copy(ids_hbm.at[pl.ds(base, CHUNK)], idxs[b])
        for j in range(NSUB):
            pltpu.async_copy(
                table_hbm.at[idxs[b].at[pl.ds(j * SUB, SUB)]],
                rows[b].at[pl.ds(j * SUB, SUB)],
                gsems[b],
            )

    def drain_gather(b):
        for j in range(NSUB):
            pltpu.make_async_copy(
                table_hbm.at[idxs[b].at[pl.ds(j * SUB, SUB)]],
                rows[b].at[pl.ds(j * SUB, SUB)],
                gsems[b],
            ).wait()

    def add_pos(b):
        def add_body(pr, c):
            for q in range(D // 16):
                pv = pos_v[pr, pl.ds(q * 16, 16)]
                for rep in range(CHUNK // L):
                    plsc.addupdate(rows[b].at[rep * L + pr, pl.ds(q * 16, 16)], pv)
            return c

        lax.fori_loop(0, L, add_body, 0)

    def fire_store(t, b):
        pltpu.async_copy(rows[b], out_hbm.at[pl.ds(wbase + t * CHUNK, CHUNK)], ssems[b])

    def drain_store(t, b):
        pltpu.make_async_copy(
            rows[b], out_hbm.at[pl.ds(wbase + t * CHUNK, CHUNK)], ssems[b]
        ).wait()

    for t in range(NBUF - 1):
        fire(t, t)

    drain_gather(0)
    add_pos(0)
    fire_store(0, 0)
    fire(NBUF - 1, NBUF - 1)

    def steady(i, c):
        t0 = NBUF * i + 1
        for k in range(NBUF):
            t = t0 + k
            b = (1 + k) % NBUF
            bn = (b + NBUF - 1) % NBUF
            drain_gather(b)
            add_pos(b)
            fire_store(t, b)
            drain_store(t - 1, bn)
            fire(t + NBUF - 1, bn)
        return c

    lax.fori_loop(0, (CHUNKS - NBUF) // NBUF, steady, 0)

    for k in range(NBUF - 1):
        t = CHUNKS - (NBUF - 1) + k
        b = t % NBUF
        drain_gather(b)
        add_pos(b)
        fire_store(t, b)
    for k in range(NBUF):
        t = CHUNKS - NBUF + k
        drain_store(t, t % NBUF)


@jax.jit
def _run(ids_flat, token_table, pos_table):
    mesh = plsc.VectorSubcoreMesh(
        core_axis_name="c", subcore_axis_name="s", num_cores=NC, num_subcores=NS
    )
    f = pl.kernel(
        _body,
        out_type=jax.ShapeDtypeStruct((ROWS, D), jnp.float32),
        mesh=mesh,
        scratch_types=[
            [pltpu.VMEM((CHUNK,), jnp.int32) for _ in range(NBUF)],
            [pltpu.VMEM((CHUNK, D), jnp.float32) for _ in range(NBUF)],
            pltpu.VMEM((L, D), jnp.float32),
            [pltpu.SemaphoreType.DMA for _ in range(NBUF)],
            [pltpu.SemaphoreType.DMA for _ in range(NBUF)],
        ],
        compiler_params=pltpu.CompilerParams(use_tc_tiling_on_sc=False),
    )
    return f(ids_flat, token_table, pos_table)


def kernel(input_ids, token_table, pos_table):
    ids_flat = input_ids.astype(jnp.int32).reshape(ROWS)
    out = _run(ids_flat, token_table, pos_table)
    return out.reshape(B, L, D)
